# baseline (speedup 1.0000x reference)
import functools

import jax
import jax.numpy as jnp
from jax import lax
from jax.experimental import pallas as pl
from jax.experimental.pallas import tpu as pltpu
from jax.experimental.pallas import tpu_sc as plsc

_N = 10000
_E = 320000
_D = 128
_NC = 2
_NS = 16
_NW = _NC * _NS
_EPW = _E // _NW
_CHUNK = 64
_NFULL = _EPW // _CHUNK
_TAIL = _EPW - _NFULL * _CHUNK
_RING = 4
_LOOK = _RING - 1
_RPT = 624
_RPT_LAST = _N - 15 * _RPT
_PTR_PAD = 10008
_LANES = 16


def _sc_aggregate_fn():
  mesh = plsc.VectorSubcoreMesh(core_axis_name="c", subcore_axis_name="s")

  @functools.partial(
      pl.kernel,
      out_type=jax.ShapeDtypeStruct((_NC, _N, _D), jnp.float32),
      mesh=mesh,
      compiler_params=pltpu.CompilerParams(needs_layout_passes=False),
      scratch_types=[
          pltpu.VMEM((_PTR_PAD,), jnp.int32),
          [pltpu.VMEM((_CHUNK, _D), jnp.float32) for _ in range(_RING)],
          [pltpu.VMEM((_CHUNK,), jnp.int32) for _ in range(_RING)],
          pltpu.VMEM((_CHUNK,), jnp.int32),
          pltpu.VMEM((_TAIL, _D), jnp.float32),
          pltpu.VMEM((_TAIL,), jnp.int32),
          pltpu.VMEM((_TAIL,), jnp.int32),
          [pltpu.SemaphoreType.DMA for _ in range(_RING)],
          [pltpu.SemaphoreType.DMA for _ in range(_RING)],
          pltpu.SemaphoreType.DMA,
          pltpu.VMEM_SHARED((_N, _D), jnp.float32),
      ],
  )
  def agg_kernel(embed_hbm, edges_hbm, ptr_hbm, zeros_hbm, out_hbm,
                 ptr_v, rows_ring, idx_ring, dst_v, rows_t, idx_t, dst_t,
                 isems, gsems, tsem, accum_sh):
    cid = lax.axis_index("c")
    sid = lax.axis_index("s")
    wid = cid * _NS + sid
    ebase = wid * _EPW

    pltpu.sync_copy(ptr_hbm, ptr_v)

    @pl.when(sid < _NS - 1)
    def _():
      pltpu.sync_copy(zeros_hbm.at[pl.ds(0, _RPT)],
                      accum_sh.at[pl.ds(sid * _RPT, _RPT)])

    @pl.when(sid == _NS - 1)
    def _():
      pltpu.sync_copy(zeros_hbm,
                      accum_sh.at[pl.ds((_NS - 1) * _RPT, _RPT_LAST)])

    plsc.subcore_barrier()

    iota = lax.iota(jnp.int32, _LANES)
    zero16 = jnp.zeros((_LANES,), jnp.int32)

    def start_idx(g, islot):
      pltpu.async_copy(edges_hbm.at[pl.ds(ebase + g * _CHUNK, _CHUNK)],
                       idx_ring[islot], isems[islot])

    def compute_dst(cbase, csize, dst_ref, a):
      cend = cbase + csize
      a_g = a
      for v in range(csize // _LANES):
        dst_ref[pl.ds(v * _LANES, _LANES)] = zero16

      def wcond(a_c):
        a_vec = zero16 + jnp.minimum(a_c, _N)
        val = jnp.max(plsc.load_gather(ptr_v, [a_vec]))
        return (a_c <= _N) & (val < cend)

      def wbody(a_c):
        k = a_c + iota
        x = plsc.load_gather(ptr_v, [jnp.minimum(k, _PTR_PAD - 1)])
        xn = plsc.load_gather(ptr_v, [jnp.minimum(k + 1, _PTR_PAD - 1)])
        in_mask = (k <= _N) & (x < cend)
        plsc.store_scatter(dst_ref, [x - cbase], k + 1,
                           mask=in_mask & (x != xn))
        cnt = plsc.all_reduce_population_count(in_mask)
        return a_c + jnp.max(cnt)

      a = lax.while_loop(wcond, wbody, a)

      cms = [plsc.cummax(dst_ref[pl.ds(v * _LANES, _LANES)])
             for v in range(csize // _LANES)]
      carry = a_g
      for v in range(csize // _LANES):
        cm = jnp.maximum(cms[v], carry)
        dst_ref[pl.ds(v * _LANES, _LANES)] = jnp.clip(cm - 1, 0, _N - 1)
        carry = jnp.max(cm)
      return a

    j0 = jnp.full((_LANES,), ebase - 1, jnp.int32)
    pos = zero16
    step = 8192
    while step >= 1:
      probe = pos + (step - 1)
      val = plsc.load_gather(ptr_v, [jnp.minimum(probe, _N)])
      pos = jnp.where((probe <= _N) & (val <= j0), pos + step, pos)
      step //= 2
    a0 = jnp.max(pos)

    for s in range(_RING):
      start_idx(s, s)
    for s in range(_LOOK):
      pltpu.make_async_copy(edges_hbm.at[pl.ds(ebase + s * _CHUNK, _CHUNK)],
                            idx_ring[s], isems[s]).wait()
      pltpu.async_copy(embed_hbm.at[idx_ring[s]], rows_ring[s], gsems[s])

    def process(g, s, a):
      a = compute_dst(ebase + g * _CHUNK, _CHUNK, dst_v, a)
      pltpu.make_async_copy(embed_hbm.at[idx_ring[s]], rows_ring[s],
                            gsems[s]).wait()
      pltpu.sync_copy(rows_ring[s], accum_sh.at[dst_v], add=True)
      return a

    def issue_next(g, s):
      t = (s + _LOOK) % _RING
      gnext = g + _LOOK
      pltpu.make_async_copy(
          edges_hbm.at[pl.ds(ebase + gnext * _CHUNK, _CHUNK)],
          idx_ring[t], isems[t]).wait()
      pltpu.async_copy(embed_hbm.at[idx_ring[t]], rows_ring[t], gsems[t])

    def outer(gg, a):
      for s in range(_RING):
        g = gg * _RING + s
        a = process(g, s, a)

        @pl.when(g + _RING < _NFULL)
        def _():
          start_idx(g + _RING, s)

        issue_next(g, s)
      return a

    a_fin = lax.fori_loop(0, (_NFULL - _RING) // _RING, outer, a0)
    a_fin = process(_NFULL - 4, 0, a_fin)
    issue_next(_NFULL - 4, 0)
    a_fin = process(_NFULL - 3, 1, a_fin)
    a_fin = process(_NFULL - 2, 2, a_fin)
    a_fin = process(_NFULL - 1, 3, a_fin)
    tbase = ebase + _NFULL * _CHUNK
    pltpu.async_copy(edges_hbm.at[pl.ds(tbase, _TAIL)], idx_t, tsem).wait()
    pltpu.async_copy(embed_hbm.at[idx_t], rows_t, tsem).wait()
    compute_dst(tbase, _TAIL, dst_t, a_fin)
    pltpu.sync_copy(rows_t, accum_sh.at[dst_t], add=True)
    plsc.subcore_barrier()

    @pl.when(sid < _NS - 1)
    def _():
      row0 = sid * _RPT
      pltpu.sync_copy(accum_sh.at[pl.ds(row0, _RPT)],
                      out_hbm.at[cid, pl.ds(row0, _RPT)])

    @pl.when(sid == _NS - 1)
    def _():
      row0 = (_NS - 1) * _RPT
      pltpu.sync_copy(accum_sh.at[pl.ds(row0, _RPT_LAST)],
                      out_hbm.at[cid, pl.ds(row0, _RPT_LAST)])

  return agg_kernel


_ROWS_BLK = 1000


def _tc_tail_kernel(a_ref, lo_ref, hi_ref, wh_ref, bh_ref, wt_ref, bt_ref,
                    o_ref):
  agg = a_ref[0] + a_ref[1]
  row = (_ROWS_BLK * pl.program_id(0)
         + lax.broadcasted_iota(jnp.int32, (_ROWS_BLK, 1), 0))
  base = jnp.where(row == 0, 0, lo_ref[...]).astype(jnp.float32)
  top = jnp.where(row == _N - 1, _E, hi_ref[...]).astype(jnp.float32)
  deg = top - base
  h = lax.dot_general(agg, wh_ref[...], (((1,), (1,)), ((), ())),
                      preferred_element_type=jnp.float32)
  h = jnp.maximum(h + deg * bh_ref[...], 0.0)
  z = lax.dot_general(h, wt_ref[...], (((1,), (1,)), ((), ())),
                      preferred_element_type=jnp.float32)
  z = jnp.maximum(z + bt_ref[...], 0.0)
  shifted = z - jnp.max(z, axis=-1, keepdims=True)
  o_ref[...] = shifted - jnp.log(
      jnp.sum(jnp.exp(shifted), axis=-1, keepdims=True))


def _tc_tail(partials, ptr_lo, ptr_hi, W_head, b_head, W_tail, b_tail):
  grid = (_N // _ROWS_BLK,)
  return pl.pallas_call(
      _tc_tail_kernel,
      grid=grid,
      in_specs=[
          pl.BlockSpec((_NC, _ROWS_BLK, _D), lambda i: (0, i, 0)),
          pl.BlockSpec((_ROWS_BLK, 1), lambda i: (i, 0)),
          pl.BlockSpec((_ROWS_BLK, 1), lambda i: (i, 0)),
          pl.BlockSpec((_D, _D), lambda i: (0, 0)),
          pl.BlockSpec((1, _D), lambda i: (0, 0)),
          pl.BlockSpec((64, _D), lambda i: (0, 0)),
          pl.BlockSpec((1, 64), lambda i: (0, 0)),
      ],
      out_specs=pl.BlockSpec((_ROWS_BLK, 64), lambda i: (i, 0)),
      out_shape=jax.ShapeDtypeStruct((_N, 64), jnp.float32),
  )(partials, ptr_lo, ptr_hi, W_head, b_head, W_tail, b_tail)


def kernel(numGroups, nodePointer, ebd_dim, numNodes, groupNodePointer,
           edgeList, embed, W_head, b_head, W_hidden, b_hidden, W_tail,
           b_tail):
  ptr = nodePointer.astype(jnp.int32)
  ptr_pad = jnp.pad(ptr, (0, _PTR_PAD - (_N + 1)),
                    constant_values=jnp.iinfo(jnp.int32).max)
  ptr_lo = lax.slice(ptr, (0,), (_N,)).reshape(_N, 1)
  ptr_hi = lax.slice(ptr, (1,), (_N + 1,)).reshape(_N, 1)

  zeros_init = jnp.zeros((_RPT_LAST, _D), jnp.float32)

  partials = _sc_aggregate_fn()(
      embed, edgeList.astype(jnp.int32), ptr_pad, zeros_init)

  return _tc_tail(partials, ptr_lo, ptr_hi, W_head,
                  b_head.reshape(1, _D), W_tail, b_tail.reshape(1, 64))


# speedup vs baseline: 1.0161x; 1.0161x over previous
import functools

import jax
import jax.numpy as jnp
from jax import lax
from jax.experimental import pallas as pl
from jax.experimental.pallas import tpu as pltpu
from jax.experimental.pallas import tpu_sc as plsc

_N = 10000
_E = 320000
_D = 128
_NC = 2
_NS = 16
_NW = _NC * _NS
_EPW = _E // _NW
_CHUNK = 64
_NFULL = _EPW // _CHUNK
_TAIL = _EPW - _NFULL * _CHUNK
_RING = 4
_LOOK = _RING - 1
_RPT = 624
_RPT_LAST = _N - 15 * _RPT
_PTR_PAD = 10008
_LANES = 16


def _sc_aggregate_fn():
  mesh = plsc.VectorSubcoreMesh(core_axis_name="c", subcore_axis_name="s")

  @functools.partial(
      pl.kernel,
      out_type=jax.ShapeDtypeStruct((_NC, _N, _D), jnp.float32),
      mesh=mesh,
      compiler_params=pltpu.CompilerParams(needs_layout_passes=False),
      scratch_types=[
          pltpu.VMEM((_PTR_PAD,), jnp.int32),
          [pltpu.VMEM((_CHUNK, _D), jnp.float32) for _ in range(_RING)],
          [pltpu.VMEM((_CHUNK,), jnp.int32) for _ in range(_RING)],
          pltpu.VMEM((_CHUNK,), jnp.int32),
          pltpu.VMEM((_TAIL, _D), jnp.float32),
          pltpu.VMEM((_TAIL,), jnp.int32),
          pltpu.VMEM((_TAIL,), jnp.int32),
          [pltpu.SemaphoreType.DMA for _ in range(_RING)],
          [pltpu.SemaphoreType.DMA for _ in range(_RING)],
          pltpu.SemaphoreType.DMA,
          pltpu.VMEM_SHARED((_N, _D), jnp.float32),
      ],
  )
  def agg_kernel(embed_hbm, edges_hbm, ptr_hbm, zeros_hbm, out_hbm,
                 ptr_v, rows_ring, idx_ring, dst_v, rows_t, idx_t, dst_t,
                 isems, gsems, tsem, accum_sh):
    cid = lax.axis_index("c")
    sid = lax.axis_index("s")
    wid = cid * _NS + sid
    ebase = wid * _EPW

    pltpu.sync_copy(ptr_hbm, ptr_v)

    @pl.when(sid < _NS - 1)
    def _():
      pltpu.sync_copy(zeros_hbm.at[pl.ds(0, _RPT)],
                      accum_sh.at[pl.ds(sid * _RPT, _RPT)])

    @pl.when(sid == _NS - 1)
    def _():
      pltpu.sync_copy(zeros_hbm,
                      accum_sh.at[pl.ds((_NS - 1) * _RPT, _RPT_LAST)])

    plsc.subcore_barrier()

    iota = lax.iota(jnp.int32, _LANES)
    zero16 = jnp.zeros((_LANES,), jnp.int32)

    def start_idx(g, islot):
      pltpu.async_copy(edges_hbm.at[pl.ds(ebase + g * _CHUNK, _CHUNK)],
                       idx_ring[islot], isems[islot])

    def compute_dst(cbase, csize, dst_ref, a):
      cend = cbase + csize
      a_g = a
      for v in range(csize // _LANES):
        dst_ref[pl.ds(v * _LANES, _LANES)] = zero16

      def wcond(a_c):
        a_vec = zero16 + jnp.minimum(a_c, _N)
        val = jnp.max(plsc.load_gather(ptr_v, [a_vec]))
        return (a_c <= _N) & (val < cend)

      def wbody(a_c):
        k = a_c + iota
        x = plsc.load_gather(ptr_v, [jnp.minimum(k, _PTR_PAD - 1)])
        xn = plsc.load_gather(ptr_v, [jnp.minimum(k + 1, _PTR_PAD - 1)])
        in_mask = (k <= _N) & (x < cend)
        plsc.store_scatter(dst_ref, [x - cbase], k + 1,
                           mask=in_mask & (x != xn))
        cnt = plsc.all_reduce_population_count(in_mask)
        return a_c + jnp.max(cnt)

      a = lax.while_loop(wcond, wbody, a)

      cms = [plsc.cummax(dst_ref[pl.ds(v * _LANES, _LANES)])
             for v in range(csize // _LANES)]
      carry = a_g
      for v in range(csize // _LANES):
        cm = jnp.maximum(cms[v], carry)
        dst_ref[pl.ds(v * _LANES, _LANES)] = jnp.clip(cm - 1, 0, _N - 1)
        carry = jnp.max(cm)
      return a

    j0 = jnp.full((_LANES,), ebase - 1, jnp.int32)
    pos = zero16
    step = 8192
    while step >= 1:
      probe = pos + (step - 1)
      val = plsc.load_gather(ptr_v, [jnp.minimum(probe, _N)])
      pos = jnp.where((probe <= _N) & (val <= j0), pos + step, pos)
      step //= 2
    a0 = jnp.max(pos)

    for s in range(_RING):
      start_idx(s, s)
    for s in range(_LOOK):
      pltpu.make_async_copy(edges_hbm.at[pl.ds(ebase + s * _CHUNK, _CHUNK)],
                            idx_ring[s], isems[s]).wait()
      pltpu.async_copy(embed_hbm.at[idx_ring[s]], rows_ring[s], gsems[s])

    def process(g, s, a):
      a = compute_dst(ebase + g * _CHUNK, _CHUNK, dst_v, a)
      pltpu.make_async_copy(embed_hbm.at[idx_ring[s]], rows_ring[s],
                            gsems[s]).wait()
      pltpu.sync_copy(rows_ring[s], accum_sh.at[dst_v], add=True)
      return a

    def issue_next(g, s):
      t = (s + _LOOK) % _RING
      gnext = g + _LOOK
      pltpu.make_async_copy(
          edges_hbm.at[pl.ds(ebase + gnext * _CHUNK, _CHUNK)],
          idx_ring[t], isems[t]).wait()
      pltpu.async_copy(embed_hbm.at[idx_ring[t]], rows_ring[t], gsems[t])

    def outer(gg, a):
      for s in range(_RING):
        g = gg * _RING + s
        a = process(g, s, a)

        @pl.when(g + _RING < _NFULL)
        def _():
          start_idx(g + _RING, s)

        issue_next(g, s)
      return a

    a_fin = lax.fori_loop(0, (_NFULL - _RING) // _RING, outer, a0)
    a_fin = process(_NFULL - 4, 0, a_fin)
    issue_next(_NFULL - 4, 0)
    a_fin = process(_NFULL - 3, 1, a_fin)
    a_fin = process(_NFULL - 2, 2, a_fin)
    a_fin = process(_NFULL - 1, 3, a_fin)
    tbase = ebase + _NFULL * _CHUNK
    pltpu.async_copy(edges_hbm.at[pl.ds(tbase, _TAIL)], idx_t, tsem).wait()
    pltpu.async_copy(embed_hbm.at[idx_t], rows_t, tsem).wait()
    compute_dst(tbase, _TAIL, dst_t, a_fin)
    pltpu.sync_copy(rows_t, accum_sh.at[dst_t], add=True)
    plsc.subcore_barrier()

    @pl.when(sid < _NS - 1)
    def _():
      row0 = sid * _RPT
      pltpu.sync_copy(accum_sh.at[pl.ds(row0, _RPT)],
                      out_hbm.at[cid, pl.ds(row0, _RPT)])

    @pl.when(sid == _NS - 1)
    def _():
      row0 = (_NS - 1) * _RPT
      pltpu.sync_copy(accum_sh.at[pl.ds(row0, _RPT_LAST)],
                      out_hbm.at[cid, pl.ds(row0, _RPT_LAST)])

  return agg_kernel


_ROWS_BLK = 1000


def _tc_tail_kernel(a_ref, deg_ref, wh_ref, bh_ref, wt_ref, bt_ref, o_ref):
  agg = a_ref[0] + a_ref[1]
  h = lax.dot_general(agg, wh_ref[...], (((1,), (1,)), ((), ())),
                      preferred_element_type=jnp.float32)
  h = jnp.maximum(h + deg_ref[...] * bh_ref[...], 0.0)
  z = lax.dot_general(h, wt_ref[...], (((1,), (1,)), ((), ())),
                      preferred_element_type=jnp.float32)
  z = jnp.maximum(z + bt_ref[...], 0.0)
  shifted = z - jnp.max(z, axis=-1, keepdims=True)
  o_ref[...] = shifted - jnp.log(
      jnp.sum(jnp.exp(shifted), axis=-1, keepdims=True))


def _tc_tail(partials, deg, W_head, b_head, W_tail, b_tail):
  grid = (_N // _ROWS_BLK,)
  return pl.pallas_call(
      _tc_tail_kernel,
      grid=grid,
      in_specs=[
          pl.BlockSpec((_NC, _ROWS_BLK, _D), lambda i: (0, i, 0)),
          pl.BlockSpec((_ROWS_BLK, 1), lambda i: (i, 0)),
          pl.BlockSpec((_D, _D), lambda i: (0, 0)),
          pl.BlockSpec((1, _D), lambda i: (0, 0)),
          pl.BlockSpec((64, _D), lambda i: (0, 0)),
          pl.BlockSpec((1, 64), lambda i: (0, 0)),
      ],
      out_specs=pl.BlockSpec((_ROWS_BLK, 64), lambda i: (i, 0)),
      out_shape=jax.ShapeDtypeStruct((_N, 64), jnp.float32),
  )(partials, deg, W_head, b_head, W_tail, b_tail)


def kernel(numGroups, nodePointer, ebd_dim, numNodes, groupNodePointer,
           edgeList, embed, W_head, b_head, W_hidden, b_hidden, W_tail,
           b_tail):
  ptr = nodePointer.astype(jnp.int32)
  ptr_pad = jnp.pad(ptr, (0, _PTR_PAD - (_N + 1)),
                    constant_values=jnp.iinfo(jnp.int32).max)
  deg = (ptr[1:] - ptr[:-1]).astype(jnp.float32)
  deg = deg.at[0].set(ptr[1].astype(jnp.float32))
  deg = deg.at[-1].set(jnp.float32(_E) - ptr[_N - 1].astype(jnp.float32))
  deg = deg.reshape(_N, 1)

  zeros_init = jnp.zeros((_RPT_LAST, _D), jnp.float32)

  partials = _sc_aggregate_fn()(
      embed, edgeList.astype(jnp.int32), ptr_pad, zeros_init)

  return _tc_tail(partials, deg, W_head, b_head.reshape(1, _D),
                  W_tail, b_tail.reshape(1, 64))


# speedup vs baseline: 1.0354x; 1.0191x over previous
import functools

import jax
import jax.numpy as jnp
from jax import lax
from jax.experimental import pallas as pl
from jax.experimental.pallas import tpu as pltpu
from jax.experimental.pallas import tpu_sc as plsc

_N = 10000
_E = 320000
_D = 128
_NC = 2
_NS = 16
_NW = _NC * _NS
_EPW = _E // _NW
_CHUNK = 64
_NFULL = _EPW // _CHUNK
_TAIL = _EPW - _NFULL * _CHUNK
_RING = 4
_LOOK = _RING - 1
_RPT = 624
_RPT_LAST = _N - 15 * _RPT
_PTR_PAD = 10008
_LANES = 16


def _sc_aggregate_fn():
  mesh = plsc.VectorSubcoreMesh(core_axis_name="c", subcore_axis_name="s")

  @functools.partial(
      pl.kernel,
      out_type=jax.ShapeDtypeStruct((_NC, _N, _D), jnp.float32),
      mesh=mesh,
      compiler_params=pltpu.CompilerParams(needs_layout_passes=False),
      scratch_types=[
          pltpu.VMEM((_PTR_PAD,), jnp.int32),
          [pltpu.VMEM((_CHUNK, _D), jnp.float32) for _ in range(_RING)],
          [pltpu.VMEM((_CHUNK,), jnp.int32) for _ in range(_RING)],
          pltpu.VMEM((_CHUNK,), jnp.int32),
          pltpu.VMEM((_TAIL, _D), jnp.float32),
          pltpu.VMEM((_TAIL,), jnp.int32),
          pltpu.VMEM((_TAIL,), jnp.int32),
          [pltpu.SemaphoreType.DMA for _ in range(_RING)],
          [pltpu.SemaphoreType.DMA for _ in range(_RING)],
          pltpu.SemaphoreType.DMA,
          pltpu.VMEM_SHARED((_N, _D), jnp.float32),
      ],
  )
  def agg_kernel(embed_hbm, edges_hbm, ptr_hbm, zeros_hbm, out_hbm,
                 ptr_v, rows_ring, idx_ring, dst_v, rows_t, idx_t, dst_t,
                 isems, gsems, tsem, accum_sh):
    cid = lax.axis_index("c")
    sid = lax.axis_index("s")
    wid = cid * _NS + sid
    ebase = wid * _EPW

    @pl.when(sid < _NS - 1)
    def _():
      pltpu.async_copy(zeros_hbm.at[pl.ds(0, _RPT)],
                       accum_sh.at[pl.ds(sid * _RPT, _RPT)], tsem)

    @pl.when(sid == _NS - 1)
    def _():
      pltpu.async_copy(zeros_hbm,
                       accum_sh.at[pl.ds((_NS - 1) * _RPT, _RPT_LAST)],
                       tsem)

    pltpu.sync_copy(ptr_hbm, ptr_v)

    iota = lax.iota(jnp.int32, _LANES)
    zero16 = jnp.zeros((_LANES,), jnp.int32)

    def start_idx(g, islot):
      pltpu.async_copy(edges_hbm.at[pl.ds(ebase + g * _CHUNK, _CHUNK)],
                       idx_ring[islot], isems[islot])

    def compute_dst(cbase, csize, dst_ref, a):
      cend = cbase + csize
      a_g = a
      for v in range(csize // _LANES):
        dst_ref[pl.ds(v * _LANES, _LANES)] = zero16

      def wcond(a_c):
        a_vec = zero16 + jnp.minimum(a_c, _N)
        val = jnp.max(plsc.load_gather(ptr_v, [a_vec]))
        return (a_c <= _N) & (val < cend)

      def wbody(a_c):
        k = a_c + iota
        x = plsc.load_gather(ptr_v, [jnp.minimum(k, _PTR_PAD - 1)])
        xn = plsc.load_gather(ptr_v, [jnp.minimum(k + 1, _PTR_PAD - 1)])
        in_mask = (k <= _N) & (x < cend)
        plsc.store_scatter(dst_ref, [x - cbase], k + 1,
                           mask=in_mask & (x != xn))
        cnt = plsc.all_reduce_population_count(in_mask)
        return a_c + jnp.max(cnt)

      a = lax.while_loop(wcond, wbody, a)

      cms = [plsc.cummax(dst_ref[pl.ds(v * _LANES, _LANES)])
             for v in range(csize // _LANES)]
      carry = a_g
      for v in range(csize // _LANES):
        cm = jnp.maximum(cms[v], carry)
        dst_ref[pl.ds(v * _LANES, _LANES)] = jnp.clip(cm - 1, 0, _N - 1)
        carry = jnp.max(cm)
      return a

    j0 = jnp.full((_LANES,), ebase - 1, jnp.int32)
    pos = zero16
    step = 8192
    while step >= 1:
      probe = pos + (step - 1)
      val = plsc.load_gather(ptr_v, [jnp.minimum(probe, _N)])
      pos = jnp.where((probe <= _N) & (val <= j0), pos + step, pos)
      step //= 2
    a0 = jnp.max(pos)

    for s in range(_RING):
      start_idx(s, s)
    for s in range(_LOOK):
      pltpu.make_async_copy(edges_hbm.at[pl.ds(ebase + s * _CHUNK, _CHUNK)],
                            idx_ring[s], isems[s]).wait()
      pltpu.async_copy(embed_hbm.at[idx_ring[s]], rows_ring[s], gsems[s])

    def process(g, s, a):
      a = compute_dst(ebase + g * _CHUNK, _CHUNK, dst_v, a)
      pltpu.make_async_copy(embed_hbm.at[idx_ring[s]], rows_ring[s],
                            gsems[s]).wait()
      pltpu.sync_copy(rows_ring[s], accum_sh.at[dst_v], add=True)
      return a

    def issue_next(g, s):
      t = (s + _LOOK) % _RING
      gnext = g + _LOOK
      pltpu.make_async_copy(
          edges_hbm.at[pl.ds(ebase + gnext * _CHUNK, _CHUNK)],
          idx_ring[t], isems[t]).wait()
      pltpu.async_copy(embed_hbm.at[idx_ring[t]], rows_ring[t], gsems[t])

    def outer(gg, a):
      for s in range(_RING):
        g = gg * _RING + s
        a = process(g, s, a)

        @pl.when(g + _RING < _NFULL)
        def _():
          start_idx(g + _RING, s)

        issue_next(g, s)
      return a

    @pl.when(sid < _NS - 1)
    def _():
      pltpu.make_async_copy(zeros_hbm.at[pl.ds(0, _RPT)],
                            accum_sh.at[pl.ds(sid * _RPT, _RPT)],
                            tsem).wait()

    @pl.when(sid == _NS - 1)
    def _():
      pltpu.make_async_copy(
          zeros_hbm, accum_sh.at[pl.ds((_NS - 1) * _RPT, _RPT_LAST)],
          tsem).wait()

    plsc.subcore_barrier()

    a_fin = lax.fori_loop(0, (_NFULL - _RING) // _RING, outer, a0)
    a_fin = process(_NFULL - 4, 0, a_fin)
    issue_next(_NFULL - 4, 0)
    a_fin = process(_NFULL - 3, 1, a_fin)
    a_fin = process(_NFULL - 2, 2, a_fin)
    a_fin = process(_NFULL - 1, 3, a_fin)
    tbase = ebase + _NFULL * _CHUNK
    pltpu.async_copy(edges_hbm.at[pl.ds(tbase, _TAIL)], idx_t, tsem).wait()
    pltpu.async_copy(embed_hbm.at[idx_t], rows_t, tsem).wait()
    compute_dst(tbase, _TAIL, dst_t, a_fin)
    pltpu.sync_copy(rows_t, accum_sh.at[dst_t], add=True)
    plsc.subcore_barrier()

    @pl.when(sid < _NS - 1)
    def _():
      row0 = sid * _RPT
      pltpu.sync_copy(accum_sh.at[pl.ds(row0, _RPT)],
                      out_hbm.at[cid, pl.ds(row0, _RPT)])

    @pl.when(sid == _NS - 1)
    def _():
      row0 = (_NS - 1) * _RPT
      pltpu.sync_copy(accum_sh.at[pl.ds(row0, _RPT_LAST)],
                      out_hbm.at[cid, pl.ds(row0, _RPT_LAST)])

  return agg_kernel


_ROWS_BLK = 1000


def _tc_tail_kernel(a_ref, deg_ref, wh_ref, bh_ref, wt_ref, bt_ref, o_ref):
  agg = a_ref[0] + a_ref[1]
  h = lax.dot_general(agg, wh_ref[...], (((1,), (1,)), ((), ())),
                      preferred_element_type=jnp.float32)
  h = jnp.maximum(h + deg_ref[...] * bh_ref[...], 0.0)
  z = lax.dot_general(h, wt_ref[...], (((1,), (1,)), ((), ())),
                      preferred_element_type=jnp.float32)
  z = jnp.maximum(z + bt_ref[...], 0.0)
  shifted = z - jnp.max(z, axis=-1, keepdims=True)
  o_ref[...] = shifted - jnp.log(
      jnp.sum(jnp.exp(shifted), axis=-1, keepdims=True))


def _tc_tail(partials, deg, W_head, b_head, W_tail, b_tail):
  grid = (_N // _ROWS_BLK,)
  return pl.pallas_call(
      _tc_tail_kernel,
      grid=grid,
      in_specs=[
          pl.BlockSpec((_NC, _ROWS_BLK, _D), lambda i: (0, i, 0)),
          pl.BlockSpec((_ROWS_BLK, 1), lambda i: (i, 0)),
          pl.BlockSpec((_D, _D), lambda i: (0, 0)),
          pl.BlockSpec((1, _D), lambda i: (0, 0)),
          pl.BlockSpec((64, _D), lambda i: (0, 0)),
          pl.BlockSpec((1, 64), lambda i: (0, 0)),
      ],
      out_specs=pl.BlockSpec((_ROWS_BLK, 64), lambda i: (i, 0)),
      out_shape=jax.ShapeDtypeStruct((_N, 64), jnp.float32),
  )(partials, deg, W_head, b_head, W_tail, b_tail)


def kernel(numGroups, nodePointer, ebd_dim, numNodes, groupNodePointer,
           edgeList, embed, W_head, b_head, W_hidden, b_hidden, W_tail,
           b_tail):
  ptr = nodePointer.astype(jnp.int32)
  ptr_pad = jnp.pad(ptr, (0, _PTR_PAD - (_N + 1)),
                    constant_values=jnp.iinfo(jnp.int32).max)
  deg = (ptr[1:] - ptr[:-1]).astype(jnp.float32)
  deg = deg.at[0].set(ptr[1].astype(jnp.float32))
  deg = deg.at[-1].set(jnp.float32(_E) - ptr[_N - 1].astype(jnp.float32))
  deg = deg.reshape(_N, 1)

  zeros_init = jnp.zeros((_RPT_LAST, _D), jnp.float32)

  partials = _sc_aggregate_fn()(
      embed, edgeList.astype(jnp.int32), ptr_pad, zeros_init)

  return _tc_tail(partials, deg, W_head, b_head.reshape(1, _D),
                  W_tail, b_tail.reshape(1, 64))
